# Initial kernel scaffold; baseline (speedup 1.0000x reference)
import jax
import jax.numpy as jnp
from jax.experimental import pallas as pl


def kernel(logits, temperatures, top_ks, top_ps, min_ps):
    raise NotImplementedError("write your pallas kernel here")


# speedup vs baseline: 11.8137x; 11.8137x over previous
import functools
import math

import jax
import jax.numpy as jnp
import numpy as np
from jax.experimental import pallas as pl
from jax.experimental.pallas import tpu as pltpu

_BLK = 1024
_NBINS = 32
_RANGE = 30.0
_NEG = -3e38
_LOG_EPS = float(np.log(np.float32(1e-38)))


def _lane_iota(shape, dim):
    return jax.lax.broadcasted_iota(jnp.int32, shape, dim)


def _gather(a, perm):
    return jnp.take_along_axis(a, perm, axis=1)


def _rank_first(va, ia, vb, ib):
    return (va > vb) | ((va == vb) & (ia < ib))


def _cmpx(v, i, lanes, partner, takes_first):
    vp = _gather(v, partner)
    ip = _gather(i, partner)
    self_first = _rank_first(v, i, vp, ip)
    keep_self = (takes_first & self_first) | ((~takes_first) & (~self_first))
    return jnp.where(keep_self, v, vp), jnp.where(keep_self, i, ip)


def _bitonic_sort128(v, i):
    lanes = _lane_iota(v.shape, 1)
    k = 2
    while k <= 128:
        j = k // 2
        while j >= 1:
            partner = lanes ^ j
            up = (lanes & k) == 0
            takes_first = (lanes < partner) == up
            v, i = _cmpx(v, i, lanes, partner, takes_first)
            j //= 2
        k *= 2
    return v, i


def _bitonic_clean128(v, i):
    lanes = _lane_iota(v.shape, 1)
    j = 64
    while j >= 1:
        partner = lanes ^ j
        takes_first = lanes < partner
        v, i = _cmpx(v, i, lanes, partner, takes_first)
        j //= 2
    return v, i


def _merge_sorted128(v1, i1, v2, i2):
    lanes = _lane_iota(v1.shape, 1)
    rev = 127 - lanes
    v2r = _gather(v2, rev)
    i2r = _gather(i2, rev)
    first = _rank_first(v1, i1, v2r, i2r)
    vm = jnp.where(first, v1, v2r)
    im = jnp.where(first, i1, i2r)
    return _bitonic_clean128(vm, im)


def _phase_kernel(x_ref, t_ref, kk_ref, tp_ref, mp_ref, g_ref,
                  ids_ref, m_ref, th_ref, w_ref,
                  M, S, Tv, Ti, C, lo, wid, S2, *, nblk, vocab):
    p = pl.program_id(0)
    b = pl.program_id(1)
    x = x_ref[...]
    t = t_ref[...]
    y = x / t
    col = b * _BLK + _lane_iota(y.shape, 1)
    y = jnp.where(col < vocab, y, _NEG)

    @pl.when(p == 0)
    def _stats_and_topk():
        bm = jnp.max(y, axis=1, keepdims=True)
        prevM = jnp.where(b == 0, jnp.full_like(bm, _NEG), M[...])
        prevS = jnp.where(b == 0, jnp.zeros_like(bm), S[...])
        Mn = jnp.maximum(prevM, bm)
        wexp = jnp.exp(y - Mn)
        M[...] = Mn
        S[...] = prevS * jnp.exp(prevM - Mn) + jnp.sum(wexp, axis=1,
                                                       keepdims=True)
        nchunk = _BLK // 128
        yr = y.reshape(y.shape[0] * nchunk, 128)
        gidx = (b * _BLK + (_lane_iota(yr.shape, 0) % nchunk) * 128
                + _lane_iota(yr.shape, 1)).astype(jnp.float32)
        sv, si = _bitonic_sort128(yr, gidx)
        while sv.shape[0] > x.shape[0]:
            half = sv.shape[0] // 2
            a_v = sv.reshape(half, 2, 128)[:, 0, :]
            a_i = si.reshape(half, 2, 128)[:, 0, :]
            b_v = sv.reshape(half, 2, 128)[:, 1, :]
            b_i = si.reshape(half, 2, 128)[:, 1, :]
            sv, si = _merge_sorted128(a_v, a_i, b_v, b_i)
        tv0 = jnp.where(b == 0, jnp.full_like(sv, _NEG), Tv[...])
        ti0 = jnp.where(b == 0, jnp.full_like(si, 3e7), Ti[...])
        nv, ni = _merge_sorted128(tv0, ti0, sv, si)
        Tv[...] = nv
        Ti[...] = ni

    @pl.when((p >= 1) & (p <= 3))
    def _hist():
        tp = tp_ref[...]

        @pl.when(b == 0)
        def _level_init():
            @pl.when(p == 1)
            def _l1():
                lo[...] = M[...] - _RANGE
                wid[...] = jnp.full_like(M[...], _RANGE)

            @pl.when(p > 1)
            def _refine():
                thr = tp * S[...]
                cnt = jnp.sum(jnp.where(C[...] > thr, 1.0, 0.0), axis=1,
                              keepdims=True)
                binw = wid[...] / _NBINS
                lo[...] = lo[...] + (cnt - 1.0) * binw
                wid[...] = binw

            C[...] = jnp.zeros_like(C[...])

        u = y - M[...]
        wexp = jnp.exp(u)
        binw = wid[...] / _NBINS
        lov = lo[...]
        cols = [
            jnp.sum(jnp.where(u >= lov + k * binw, wexp, 0.0), axis=1,
                    keepdims=True)
            for k in range(_NBINS)
        ]
        C[...] = C[...] + jnp.concatenate(cols, axis=1)

    @pl.when(p == 4)
    def _mass_and_sample():
        @pl.when(b == 0)
        def _theta():
            thr = tp_ref[...] * S[...]
            cnt = jnp.sum(jnp.where(C[...] > thr, 1.0, 0.0), axis=1,
                          keepdims=True)
            binw = wid[...] / _NBINS
            th_ref[...] = jnp.minimum(lo[...] + (cnt - 1.0) * binw,
                                      jnp.zeros_like(binw))
            S2[...] = jnp.zeros_like(S2[...])

        u = y - M[...]
        wexp = jnp.exp(u)
        S2[...] = S2[...] + jnp.sum(jnp.where(u >= th_ref[...], wexp, 0.0),
                                    axis=1, keepdims=True)

        @pl.when(b == nblk - 1)
        def _finalize():
            m_ref[...] = M[...]
            w_ref[...] = S2[...]
            p128 = jnp.exp(Tv[...] - M[...]) / S[...]
            idx = Ti[...]
            lanes = _lane_iota(p128.shape, 1)
            for parity in (0, 1, 0, 1, 0, 1):
                if parity == 0:
                    partner = lanes ^ 1
                else:
                    partner = jnp.clip(
                        lanes + jnp.where(lanes % 2 == 1, 1, -1), 0, 127)
                pp = _gather(p128, partner)
                ip = _gather(idx, partner)
                eq = p128 == pp
                left = lanes < partner
                idx = jnp.where(eq & left, jnp.minimum(idx, ip),
                                jnp.where(eq & (~left),
                                          jnp.maximum(idx, ip), idx))
            cs = p128
            for d in (1, 2, 4, 8, 16, 32, 64):
                shifted = _gather(cs, jnp.maximum(lanes - d, 0))
                cs = cs + jnp.where(lanes >= d, shifted, 0.0)
            p64 = p128[:, :64]
            i64 = idx[:, :64]
            cs64 = cs[:, :64]
            ar = _lane_iota(p64.shape, 1).astype(jnp.float32)
            ps1 = jnp.where(ar < kk_ref[...], p64, 0.0)
            ps2 = jnp.where(cs64 - ps1 > tp_ref[...], 0.0, ps1)
            ps3 = jnp.where(ps2 < ps2[:, 0:1] * mp_ref[...], 0.0, ps2)
            logp = jnp.where(ps3 > 0.0,
                             jnp.log(jnp.maximum(ps3, 1e-38)), -1e30)
            sc = logp + g_ref[...]
            vm = jnp.max(sc, axis=1, keepdims=True)
            pos = jnp.min(jnp.where(sc == vm, ar, 1000.0), axis=1,
                          keepdims=True)
            ids_ref[...] = jnp.sum(jnp.where(ar == pos, i64, 0.0), axis=1,
                                   keepdims=True).astype(jnp.int32)


def _out_kernel(x_ref, t_ref, m_ref, th_ref, w_ref, mv_ref, o_ref):
    u = x_ref[...] / t_ref[...] - m_ref[...]
    val = jnp.log(jnp.maximum(jnp.exp(u) / w_ref[...], 1e-38))
    o_ref[...] = jnp.where(u >= th_ref[...], val, mv_ref[...])


@jax.jit
def kernel(logits, temperatures, top_ks, top_ps, min_ps):
    bsz, vocab = logits.shape
    nblk = math.ceil(vocab / _BLK)
    gum = jax.random.gumbel(jax.random.key(123), (bsz, vocab),
                            jnp.float32)[:, :64]
    kk = top_ks.reshape(bsz, 1).astype(jnp.float32)
    tp = top_ps.reshape(bsz, 1)
    mp = min_ps.reshape(bsz, 1)

    small = lambda p, b: (0, 0)
    col1 = pl.BlockSpec((bsz, 1), small)
    ids, m, th, w = pl.pallas_call(
        functools.partial(_phase_kernel, nblk=nblk, vocab=vocab),
        grid=(5, nblk),
        in_specs=[
            pl.BlockSpec((bsz, _BLK), lambda p, b: (0, b)),
            col1, col1, col1, col1,
            pl.BlockSpec((bsz, 64), small),
        ],
        out_specs=[col1, col1, col1, col1],
        out_shape=[
            jax.ShapeDtypeStruct((bsz, 1), jnp.int32),
            jax.ShapeDtypeStruct((bsz, 1), jnp.float32),
            jax.ShapeDtypeStruct((bsz, 1), jnp.float32),
            jax.ShapeDtypeStruct((bsz, 1), jnp.float32),
        ],
        scratch_shapes=[
            pltpu.VMEM((bsz, 1), jnp.float32),
            pltpu.VMEM((bsz, 1), jnp.float32),
            pltpu.VMEM((bsz, 128), jnp.float32),
            pltpu.VMEM((bsz, 128), jnp.float32),
            pltpu.VMEM((bsz, _NBINS), jnp.float32),
            pltpu.VMEM((bsz, 1), jnp.float32),
            pltpu.VMEM((bsz, 1), jnp.float32),
            pltpu.VMEM((bsz, 1), jnp.float32),
        ],
    )(logits, temperatures, kk, tp, mp, gum)

    mask_val = jnp.maximum(
        jnp.log(jnp.maximum(logits[:, :1] * 0.0, 1e-38)),
        jnp.finfo(jnp.float32).min)

    colb = pl.BlockSpec((bsz, 1), lambda b: (0, 0))
    logprobs = pl.pallas_call(
        _out_kernel,
        grid=(nblk,),
        in_specs=[
            pl.BlockSpec((bsz, _BLK), lambda b: (0, b)),
            colb, colb, colb, colb, colb,
        ],
        out_specs=pl.BlockSpec((bsz, _BLK), lambda b: (0, b)),
        out_shape=jax.ShapeDtypeStruct((bsz, vocab), jnp.float32),
    )(logits, temperatures, m, th, w, mask_val)
    return ids.reshape(-1), logprobs


# speedup vs baseline: 11.9955x; 1.0154x over previous
import functools
import math

import jax
import jax.numpy as jnp
import numpy as np
from jax.experimental import pallas as pl
from jax.experimental.pallas import tpu as pltpu

_BLK = 1024
_NBINS = 32
_RANGE = 30.0
_NEG = -3e38
_LOG_EPS = float(np.log(np.float32(1e-38)))


def _lane_iota(shape, dim):
    return jax.lax.broadcasted_iota(jnp.int32, shape, dim)


def _gather(a, perm):
    return jnp.take_along_axis(a, perm, axis=1)


def _rank_first(va, ia, vb, ib):
    return (va > vb) | ((va == vb) & (ia < ib))


def _cmpx(v, i, lanes, partner, takes_first):
    vp = _gather(v, partner)
    ip = _gather(i, partner)
    self_first = _rank_first(v, i, vp, ip)
    keep_self = (takes_first & self_first) | ((~takes_first) & (~self_first))
    return jnp.where(keep_self, v, vp), jnp.where(keep_self, i, ip)


def _bitonic_sort128(v, i):
    lanes = _lane_iota(v.shape, 1)
    k = 2
    while k <= 128:
        j = k // 2
        while j >= 1:
            partner = lanes ^ j
            up = (lanes & k) == 0
            takes_first = (lanes < partner) == up
            v, i = _cmpx(v, i, lanes, partner, takes_first)
            j //= 2
        k *= 2
    return v, i


def _bitonic_clean128(v, i):
    lanes = _lane_iota(v.shape, 1)
    j = 64
    while j >= 1:
        partner = lanes ^ j
        takes_first = lanes < partner
        v, i = _cmpx(v, i, lanes, partner, takes_first)
        j //= 2
    return v, i


def _merge_sorted128(v1, i1, v2, i2):
    lanes = _lane_iota(v1.shape, 1)
    rev = 127 - lanes
    v2r = _gather(v2, rev)
    i2r = _gather(i2, rev)
    first = _rank_first(v1, i1, v2r, i2r)
    vm = jnp.where(first, v1, v2r)
    im = jnp.where(first, i1, i2r)
    return _bitonic_clean128(vm, im)


def _phase_kernel(x_ref, t_ref, kk_ref, tp_ref, mp_ref, g_ref,
                  ids_ref, m_ref, th_ref, w_ref,
                  M, S, Tv, Ti, C, lo, wid, *, nblk, vocab):
    p = pl.program_id(0)
    b = pl.program_id(1)
    x = x_ref[...]
    t = t_ref[...]
    y = x / t
    col = b * _BLK + _lane_iota(y.shape, 1)
    y = jnp.where(col < vocab, y, _NEG)

    @pl.when(p == 0)
    def _stats_and_topk():
        bm = jnp.max(y, axis=1, keepdims=True)
        prevM = jnp.where(b == 0, jnp.full_like(bm, _NEG), M[...])
        prevS = jnp.where(b == 0, jnp.zeros_like(bm), S[...])
        Mn = jnp.maximum(prevM, bm)
        wexp = jnp.exp(y - Mn)
        M[...] = Mn
        S[...] = prevS * jnp.exp(prevM - Mn) + jnp.sum(wexp, axis=1,
                                                       keepdims=True)
        nchunk = _BLK // 128
        yr = y.reshape(y.shape[0] * nchunk, 128)
        gidx = (b * _BLK + (_lane_iota(yr.shape, 0) % nchunk) * 128
                + _lane_iota(yr.shape, 1)).astype(jnp.float32)
        sv, si = _bitonic_sort128(yr, gidx)
        while sv.shape[0] > x.shape[0]:
            half = sv.shape[0] // 2
            a_v = sv.reshape(half, 2, 128)[:, 0, :]
            a_i = si.reshape(half, 2, 128)[:, 0, :]
            b_v = sv.reshape(half, 2, 128)[:, 1, :]
            b_i = si.reshape(half, 2, 128)[:, 1, :]
            sv, si = _merge_sorted128(a_v, a_i, b_v, b_i)
        tv0 = jnp.where(b == 0, jnp.full_like(sv, _NEG), Tv[...])
        ti0 = jnp.where(b == 0, jnp.full_like(si, 3e7), Ti[...])
        nv, ni = _merge_sorted128(tv0, ti0, sv, si)
        Tv[...] = nv
        Ti[...] = ni

    @pl.when((p >= 1) & (p <= 3))
    def _hist():
        tp = tp_ref[...]

        @pl.when(b == 0)
        def _level_init():
            @pl.when(p == 1)
            def _l1():
                lo[...] = M[...] - _RANGE
                wid[...] = jnp.full_like(M[...], _RANGE)

            @pl.when(p > 1)
            def _refine():
                thr = tp * S[...]
                cnt = jnp.sum(jnp.where(C[...] > thr, 1.0, 0.0), axis=1,
                              keepdims=True)
                binw = wid[...] / _NBINS
                lo[...] = lo[...] + (cnt - 1.0) * binw
                wid[...] = binw

            C[...] = jnp.zeros_like(C[...])

        u = y - M[...]
        wexp = jnp.exp(u)
        binw = wid[...] / _NBINS
        lov = lo[...]
        cols = [
            jnp.sum(jnp.where(u >= lov + k * binw, wexp, 0.0), axis=1,
                    keepdims=True)
            for k in range(_NBINS)
        ]
        C[...] = C[...] + jnp.concatenate(cols, axis=1)

        @pl.when((p == 3) & (b == nblk - 1))
        def _cutoff():
            thr = tp * S[...]
            cnt = jnp.sum(jnp.where(C[...] > thr, 1.0, 0.0), axis=1,
                          keepdims=True)
            bw3 = wid[...] / _NBINS
            th_ref[...] = jnp.minimum(lo[...] + (cnt - 1.0) * bw3,
                                      jnp.zeros_like(bw3))
            kidx = _lane_iota(C[...].shape, 1).astype(jnp.float32)
            w_ref[...] = jnp.sum(
                jnp.where(kidx == cnt - 1.0, C[...], 0.0), axis=1,
                keepdims=True)
            m_ref[...] = M[...]

    @pl.when((p == 3) & (b == nblk - 1))
    def _sample():
        p128 = jnp.exp(Tv[...] - M[...]) / S[...]
        idx = Ti[...]
        lanes = _lane_iota(p128.shape, 1)
        for parity in (0, 1, 0, 1, 0, 1):
            if parity == 0:
                partner = lanes ^ 1
            else:
                partner = jnp.clip(
                    lanes + jnp.where(lanes % 2 == 1, 1, -1), 0, 127)
            pp = _gather(p128, partner)
            ip = _gather(idx, partner)
            eq = p128 == pp
            left = lanes < partner
            idx = jnp.where(eq & left, jnp.minimum(idx, ip),
                            jnp.where(eq & (~left),
                                      jnp.maximum(idx, ip), idx))
        cs = p128
        for d in (1, 2, 4, 8, 16, 32, 64):
            shifted = _gather(cs, jnp.maximum(lanes - d, 0))
            cs = cs + jnp.where(lanes >= d, shifted, 0.0)
        p64 = p128[:, :64]
        i64 = idx[:, :64]
        cs64 = cs[:, :64]
        ar = _lane_iota(p64.shape, 1).astype(jnp.float32)
        ps1 = jnp.where(ar < kk_ref[...], p64, 0.0)
        ps2 = jnp.where(cs64 - ps1 > tp_ref[...], 0.0, ps1)
        ps3 = jnp.where(ps2 < ps2[:, 0:1] * mp_ref[...], 0.0, ps2)
        logp = jnp.where(ps3 > 0.0,
                         jnp.log(jnp.maximum(ps3, 1e-38)), -1e30)
        sc = logp + g_ref[...]
        vm = jnp.max(sc, axis=1, keepdims=True)
        pos = jnp.min(jnp.where(sc == vm, ar, 1000.0), axis=1,
                      keepdims=True)
        ids_ref[...] = jnp.sum(jnp.where(ar == pos, i64, 0.0), axis=1,
                               keepdims=True).astype(jnp.int32)


def _out_kernel(x_ref, t_ref, m_ref, th_ref, w_ref, mv_ref, o_ref):
    u = x_ref[...] / t_ref[...] - m_ref[...]
    val = jnp.log(jnp.maximum(jnp.exp(u) / w_ref[...], 1e-38))
    o_ref[...] = jnp.where(u >= th_ref[...], val, mv_ref[...])


@jax.jit
def kernel(logits, temperatures, top_ks, top_ps, min_ps):
    bsz, vocab = logits.shape
    nblk = math.ceil(vocab / _BLK)
    gum = jax.random.gumbel(jax.random.key(123), (bsz, vocab),
                            jnp.float32)[:, :64]
    kk = top_ks.reshape(bsz, 1).astype(jnp.float32)
    tp = top_ps.reshape(bsz, 1)
    mp = min_ps.reshape(bsz, 1)

    small = lambda p, b: (0, 0)
    col1 = pl.BlockSpec((bsz, 1), small)
    ids, m, th, w = pl.pallas_call(
        functools.partial(_phase_kernel, nblk=nblk, vocab=vocab),
        grid=(4, nblk),
        in_specs=[
            pl.BlockSpec((bsz, _BLK), lambda p, b: (0, b)),
            col1, col1, col1, col1,
            pl.BlockSpec((bsz, 64), small),
        ],
        out_specs=[col1, col1, col1, col1],
        out_shape=[
            jax.ShapeDtypeStruct((bsz, 1), jnp.int32),
            jax.ShapeDtypeStruct((bsz, 1), jnp.float32),
            jax.ShapeDtypeStruct((bsz, 1), jnp.float32),
            jax.ShapeDtypeStruct((bsz, 1), jnp.float32),
        ],
        scratch_shapes=[
            pltpu.VMEM((bsz, 1), jnp.float32),
            pltpu.VMEM((bsz, 1), jnp.float32),
            pltpu.VMEM((bsz, 128), jnp.float32),
            pltpu.VMEM((bsz, 128), jnp.float32),
            pltpu.VMEM((bsz, _NBINS), jnp.float32),
            pltpu.VMEM((bsz, 1), jnp.float32),
            pltpu.VMEM((bsz, 1), jnp.float32),
        ],
    )(logits, temperatures, kk, tp, mp, gum)

    mask_val = jnp.maximum(
        jnp.log(jnp.maximum(logits[:, :1] * 0.0, 1e-38)),
        jnp.finfo(jnp.float32).min)

    colb = pl.BlockSpec((bsz, 1), lambda b: (0, 0))
    logprobs = pl.pallas_call(
        _out_kernel,
        grid=(nblk,),
        in_specs=[
            pl.BlockSpec((bsz, _BLK), lambda b: (0, b)),
            colb, colb, colb, colb, colb,
        ],
        out_specs=pl.BlockSpec((bsz, _BLK), lambda b: (0, b)),
        out_shape=jax.ShapeDtypeStruct((bsz, vocab), jnp.float32),
    )(logits, temperatures, m, th, w, mask_val)
    return ids.reshape(-1), logprobs
